# Initial kernel scaffold; baseline (speedup 1.0000x reference)
import jax
import jax.numpy as jnp
from jax.experimental import pallas as pl


def kernel(edge_index, emb, W1, b1, W2, b2):
    raise NotImplementedError("write your pallas kernel here")


# speedup vs baseline: 13.2356x; 13.2356x over previous
import functools

import jax
import jax.numpy as jnp
from jax import lax
from jax.experimental import pallas as pl
from jax.experimental.pallas import tpu as pltpu
from jax.experimental.pallas import tpu_sc as plsc

N = 10000
E = 320000
D = 128
NP = 10240
NC = 2
NS = 16
NW = NC * NS
EPW = E // NW
K = 80
NCHUNK = EPW // K
RPS = NP // NS

_mesh = plsc.VectorSubcoreMesh(core_axis_name="c", subcore_axis_name="s")


@functools.partial(
    pl.kernel,
    mesh=_mesh,
    out_type=jax.ShapeDtypeStruct((NC, NP), jnp.float32),
    scratch_types=[
        pltpu.VMEM((K,), jnp.int32),
        pltpu.VMEM((K,), jnp.float32),
        pltpu.VMEM_SHARED((NP,), jnp.float32),
    ],
)
def _sc_deg(dst_hbm, zeros_hbm, out_hbm, dst_v, ones_v, acc_sh):
    c = lax.axis_index("c")
    s = lax.axis_index("s")
    wid = s * NC + c
    for i in range(K // 16):
        ones_v[pl.ds(i * 16, 16)] = jnp.ones((16,), jnp.float32)
    pltpu.sync_copy(zeros_hbm.at[pl.ds(s * RPS, RPS)],
                    acc_sh.at[pl.ds(s * RPS, RPS)])
    plsc.subcore_barrier()
    ebase = wid * EPW

    def body(j, carry):
        base = pl.multiple_of(ebase + j * K, 8)
        pltpu.sync_copy(dst_hbm.at[pl.ds(base, K)], dst_v)
        pltpu.sync_copy(ones_v, acc_sh.at[dst_v], add=True)
        return carry

    lax.fori_loop(0, NCHUNK, body, 0)
    plsc.subcore_barrier()
    pltpu.sync_copy(acc_sh.at[pl.ds(s * RPS, RPS)],
                    out_hbm.at[c, pl.ds(s * RPS, RPS)])


@functools.partial(
    pl.kernel,
    mesh=_mesh,
    out_type=jax.ShapeDtypeStruct((NC, NP, D), jnp.float32),
    scratch_types=[
        pltpu.VMEM((K,), jnp.int32),
        pltpu.VMEM((K,), jnp.int32),
        pltpu.VMEM((K, D), jnp.float32),
        pltpu.VMEM_SHARED((NP, D), jnp.float32),
        pltpu.SemaphoreType.DMA,
    ],
)
def _sc_agg(x_hbm, src_hbm, dst_hbm, zeros_hbm, out_hbm,
            src_v, dst_v, rows_v, acc_sh, sem):
    c = lax.axis_index("c")
    s = lax.axis_index("s")
    wid = s * NC + c
    pltpu.sync_copy(zeros_hbm.at[pl.ds(s * RPS, RPS)],
                    acc_sh.at[pl.ds(s * RPS, RPS)])
    plsc.subcore_barrier()
    ebase = wid * EPW

    def body(j, carry):
        base = pl.multiple_of(ebase + j * K, 8)
        pltpu.sync_copy(src_hbm.at[pl.ds(base, K)], src_v)
        pltpu.sync_copy(dst_hbm.at[pl.ds(base, K)], dst_v)
        pltpu.async_copy(x_hbm.at[src_v], rows_v, sem).wait()
        pltpu.sync_copy(rows_v, acc_sh.at[dst_v], add=True)
        return carry

    lax.fori_loop(0, NCHUNK, body, 0)
    plsc.subcore_barrier()
    pltpu.sync_copy(acc_sh.at[pl.ds(s * RPS, RPS)],
                    out_hbm.at[c, pl.ds(s * RPS, RPS)])


RB = 2048


def _prep_body(deg_ref, emb_ref, xp_ref, dinv_ref):
    deg = deg_ref[0, :] + deg_ref[1, :] + 1.0
    dinv = lax.rsqrt(deg)[:, None]
    dinv_ref[...] = jnp.broadcast_to(dinv, (RB, D))
    xp_ref[...] = emb_ref[...] * dinv


_prep = pl.pallas_call(
    _prep_body,
    grid=(NP // RB,),
    in_specs=[
        pl.BlockSpec((NC, RB), lambda i: (0, i)),
        pl.BlockSpec((RB, D), lambda i: (i, 0)),
    ],
    out_specs=[
        pl.BlockSpec((RB, D), lambda i: (i, 0)),
        pl.BlockSpec((RB, D), lambda i: (i, 0)),
    ],
    out_shape=[
        jax.ShapeDtypeStruct((NP, D), jnp.float32),
        jax.ShapeDtypeStruct((NP, D), jnp.float32),
    ],
)


def _layer_body(relu, p_ref, xp_ref, dinv_ref, w_ref, b_ref, out_ref):
    z = (p_ref[0] + p_ref[1] + xp_ref[...]) * dinv_ref[...]
    h = jnp.dot(z, w_ref[...], preferred_element_type=jnp.float32) + b_ref[...]
    if relu:
        h = jnp.maximum(h, 0.0) * dinv_ref[...]
    out_ref[...] = h


def _make_layer(relu):
    return pl.pallas_call(
        functools.partial(_layer_body, relu),
        grid=(NP // RB,),
        in_specs=[
            pl.BlockSpec((NC, RB, D), lambda i: (0, i, 0)),
            pl.BlockSpec((RB, D), lambda i: (i, 0)),
            pl.BlockSpec((RB, D), lambda i: (i, 0)),
            pl.BlockSpec((D, D), lambda i: (0, 0)),
            pl.BlockSpec((1, D), lambda i: (0, 0)),
        ],
        out_specs=pl.BlockSpec((RB, D), lambda i: (i, 0)),
        out_shape=jax.ShapeDtypeStruct((NP, D), jnp.float32),
    )


_layer1 = _make_layer(True)
_layer2 = _make_layer(False)


def kernel(edge_index, emb, W1, b1, W2, b2):
    ei = edge_index.astype(jnp.int32)
    src = ei[0]
    dst = ei[1]
    emb_p = jnp.pad(emb, ((0, NP - N), (0, 0)))
    z2d = jnp.zeros((NP, D), jnp.float32)
    z1d = jnp.zeros((NP,), jnp.float32)

    deg_p = _sc_deg(dst, z1d)
    x0p, dinvb = _prep(deg_p, emb_p)
    p1 = _sc_agg(x0p, src, dst, z2d)
    x1p = _layer1(p1, x0p, dinvb, W1, b1.reshape(1, D))
    p2 = _sc_agg(x1p, src, dst, z2d)
    h2 = _layer2(p2, x1p, dinvb, W2, b2.reshape(1, D))
    return h2[:N]


# speedup vs baseline: 14.2174x; 1.0742x over previous
import functools

import jax
import jax.numpy as jnp
from jax import lax
from jax.experimental import pallas as pl
from jax.experimental.pallas import tpu as pltpu
from jax.experimental.pallas import tpu_sc as plsc

N = 10000
E = 320000
D = 128
NP = 10240
NC = 2
NS = 16
NW = NC * NS
K = 128
NCHUNK = -(-(E // NW) // K)
EPAD = NW * NCHUNK * K
DUMMY = NP - 1
RPS = NP // NS

_mesh = plsc.VectorSubcoreMesh(core_axis_name="c", subcore_axis_name="s")


@functools.partial(
    pl.kernel,
    mesh=_mesh,
    out_type=jax.ShapeDtypeStruct((NC, NP), jnp.float32),
    scratch_types=[
        pltpu.VMEM((NCHUNK, K), jnp.int32),
        pltpu.VMEM((K,), jnp.float32),
        pltpu.VMEM_SHARED((NP,), jnp.float32),
        pltpu.SemaphoreType.DMA,
    ],
)
def _sc_deg(dst_hbm, zeros_hbm, out_hbm, dst_v, ones_v, acc_sh, sem):
    c = lax.axis_index("c")
    s = lax.axis_index("s")
    wid = s * NC + c
    for i in range(K // 16):
        ones_v[pl.ds(i * 16, 16)] = jnp.ones((16,), jnp.float32)
    pltpu.sync_copy(dst_hbm.at[wid], dst_v)
    pltpu.sync_copy(zeros_hbm.at[pl.ds(s * RPS, RPS)],
                    acc_sh.at[pl.ds(s * RPS, RPS)])
    plsc.subcore_barrier()

    DEPTH = 4

    def body(j, carry):
        pltpu.async_copy(ones_v, acc_sh.at[dst_v.at[j]], sem, add=True)

        @pl.when(j >= DEPTH)
        def _():
            pltpu.make_async_copy(zeros_hbm.at[pl.ds(0, K)], ones_v, sem).wait()

        return carry

    lax.fori_loop(0, NCHUNK, body, 0)

    def drain(j, carry):
        pltpu.make_async_copy(zeros_hbm.at[pl.ds(0, K)], ones_v, sem).wait()
        return carry

    lax.fori_loop(0, DEPTH, drain, 0)
    plsc.subcore_barrier()
    pltpu.sync_copy(acc_sh.at[pl.ds(s * RPS, RPS)],
                    out_hbm.at[c, pl.ds(s * RPS, RPS)])


assert NCHUNK % 2 == 1 and NCHUNK >= 5
_M = (NCHUNK - 3) // 2


@functools.partial(
    pl.kernel,
    mesh=_mesh,
    out_type=jax.ShapeDtypeStruct((NC, NP, D), jnp.float32),
    scratch_types=[
        pltpu.VMEM((2, K), jnp.int32),
        pltpu.VMEM((2, K), jnp.int32),
        pltpu.VMEM((K, D), jnp.float32),
        pltpu.VMEM((K, D), jnp.float32),
        pltpu.VMEM_SHARED((NP, D), jnp.float32),
        pltpu.SemaphoreType.DMA,
        pltpu.SemaphoreType.DMA,
        pltpu.SemaphoreType.DMA,
        pltpu.SemaphoreType.DMA,
    ],
)
def _sc_agg(x_hbm, sd_hbm, zeros_hbm, out_hbm,
            idx_a, idx_b, buf_a, buf_b, acc_sh, sem_a, sem_b, sem_ia, sem_ib):
    c = lax.axis_index("c")
    s = lax.axis_index("s")
    wid = s * NC + c

    def wait_buf(buf, sem):
        pltpu.make_async_copy(x_hbm.at[pl.ds(0, K)], buf, sem).wait()

    def wait_idx(idx, sem):
        pltpu.make_async_copy(sd_hbm.at[0, 0], idx, sem).wait()

    pltpu.sync_copy(sd_hbm.at[wid, 0], idx_a)
    pltpu.sync_copy(sd_hbm.at[wid, 1], idx_b)
    pltpu.sync_copy(zeros_hbm.at[pl.ds(s * RPS, RPS)],
                    acc_sh.at[pl.ds(s * RPS, RPS)])
    plsc.subcore_barrier()

    pltpu.async_copy(x_hbm.at[idx_a.at[0]], buf_a, sem_a)
    pltpu.async_copy(x_hbm.at[idx_b.at[0]], buf_b, sem_b)

    def body(i, carry):
        ja = 2 * i
        jb = 2 * i + 1
        wait_buf(buf_a, sem_a)
        pltpu.sync_copy(buf_a, acc_sh.at[idx_a.at[1]], add=True)
        pltpu.async_copy(sd_hbm.at[wid, ja + 2], idx_a, sem_ia)
        wait_buf(buf_b, sem_b)
        pltpu.sync_copy(buf_b, acc_sh.at[idx_b.at[1]], add=True)
        wait_idx(idx_a, sem_ia)
        pltpu.async_copy(x_hbm.at[idx_a.at[0]], buf_a, sem_a)
        pltpu.async_copy(sd_hbm.at[wid, jb + 2], idx_b, sem_ib)
        wait_idx(idx_b, sem_ib)
        pltpu.async_copy(x_hbm.at[idx_b.at[0]], buf_b, sem_b)
        return carry

    lax.fori_loop(0, _M, body, 0)

    wait_buf(buf_a, sem_a)
    pltpu.sync_copy(buf_a, acc_sh.at[idx_a.at[1]], add=True)
    pltpu.async_copy(sd_hbm.at[wid, NCHUNK - 1], idx_a, sem_ia)
    wait_buf(buf_b, sem_b)
    pltpu.sync_copy(buf_b, acc_sh.at[idx_b.at[1]], add=True)
    wait_idx(idx_a, sem_ia)
    pltpu.async_copy(x_hbm.at[idx_a.at[0]], buf_a, sem_a)
    wait_buf(buf_a, sem_a)
    pltpu.sync_copy(buf_a, acc_sh.at[idx_a.at[1]], add=True)

    plsc.subcore_barrier()
    pltpu.sync_copy(acc_sh.at[pl.ds(s * RPS, RPS)],
                    out_hbm.at[c, pl.ds(s * RPS, RPS)])


RB = 2048


def _prep_body(deg_ref, emb_ref, xp_ref, dinv_ref):
    deg = deg_ref[0, :] + deg_ref[1, :] + 1.0
    dinv = lax.rsqrt(deg)[:, None]
    dinv_ref[...] = jnp.broadcast_to(dinv, (RB, D))
    xp_ref[...] = emb_ref[...] * dinv


_prep = pl.pallas_call(
    _prep_body,
    grid=(NP // RB,),
    in_specs=[
        pl.BlockSpec((NC, RB), lambda i: (0, i)),
        pl.BlockSpec((RB, D), lambda i: (i, 0)),
    ],
    out_specs=[
        pl.BlockSpec((RB, D), lambda i: (i, 0)),
        pl.BlockSpec((RB, D), lambda i: (i, 0)),
    ],
    out_shape=[
        jax.ShapeDtypeStruct((NP, D), jnp.float32),
        jax.ShapeDtypeStruct((NP, D), jnp.float32),
    ],
)


def _layer_body(relu, p_ref, xp_ref, dinv_ref, w_ref, b_ref, out_ref):
    z = (p_ref[0] + p_ref[1] + xp_ref[...]) * dinv_ref[...]
    h = jnp.dot(z, w_ref[...], preferred_element_type=jnp.float32) + b_ref[...]
    if relu:
        h = jnp.maximum(h, 0.0) * dinv_ref[...]
    out_ref[...] = h


def _make_layer(relu):
    return pl.pallas_call(
        functools.partial(_layer_body, relu),
        grid=(NP // RB,),
        in_specs=[
            pl.BlockSpec((NC, RB, D), lambda i: (0, i, 0)),
            pl.BlockSpec((RB, D), lambda i: (i, 0)),
            pl.BlockSpec((RB, D), lambda i: (i, 0)),
            pl.BlockSpec((D, D), lambda i: (0, 0)),
            pl.BlockSpec((1, D), lambda i: (0, 0)),
        ],
        out_specs=pl.BlockSpec((RB, D), lambda i: (i, 0)),
        out_shape=jax.ShapeDtypeStruct((NP, D), jnp.float32),
    )


_layer1 = _make_layer(True)
_layer2 = _make_layer(False)


def kernel(edge_index, emb, W1, b1, W2, b2):
    ei = edge_index.astype(jnp.int32)
    src = jnp.concatenate([ei[0], jnp.zeros((EPAD - E,), jnp.int32)])
    dst = jnp.concatenate([ei[1], jnp.full((EPAD - E,), DUMMY, jnp.int32)])
    src3 = src.reshape(NW, NCHUNK, K)
    dst3 = dst.reshape(NW, NCHUNK, K)
    sd = jnp.stack([src3, dst3], axis=2)
    emb_p = jnp.pad(emb, ((0, NP - N), (0, 0)))
    z2d = jnp.zeros((NP, D), jnp.float32)
    z1d = jnp.zeros((NP,), jnp.float32)

    deg_p = _sc_deg(dst3, z1d)
    x0p, dinvb = _prep(deg_p, emb_p)
    p1 = _sc_agg(x0p, sd, z2d)
    x1p = _layer1(p1, x0p, dinvb, W1, b1.reshape(1, D))
    p2 = _sc_agg(x1p, sd, z2d)
    h2 = _layer2(p2, x1p, dinvb, W2, b2.reshape(1, D))
    return h2[:N]


# speedup vs baseline: 16.1932x; 1.1390x over previous
import functools

import jax
import jax.numpy as jnp
from jax import lax
from jax.experimental import pallas as pl
from jax.experimental.pallas import tpu as pltpu
from jax.experimental.pallas import tpu_sc as plsc

N = 10000
E = 320000
D = 128
NP = 10240
NC = 2
NS = 16
NW = NC * NS
K = 128
NCHUNK = -(-(E // NW) // K)
EPAD = NW * NCHUNK * K
DUMMY = NP - 1
RPS = NP // NS

_mesh = plsc.VectorSubcoreMesh(core_axis_name="c", subcore_axis_name="s")


@functools.partial(
    pl.kernel,
    mesh=_mesh,
    out_type=jax.ShapeDtypeStruct((NC, NP), jnp.float32),
    scratch_types=[
        pltpu.VMEM((NCHUNK, K), jnp.int32),
        pltpu.VMEM((K,), jnp.float32),
        pltpu.VMEM_SHARED((NP,), jnp.float32),
        pltpu.SemaphoreType.DMA,
    ],
)
def _sc_deg(dst_hbm, zeros_hbm, out_hbm, dst_v, ones_v, acc_sh, sem):
    c = lax.axis_index("c")
    s = lax.axis_index("s")
    wid = s * NC + c
    for i in range(K // 16):
        ones_v[pl.ds(i * 16, 16)] = jnp.ones((16,), jnp.float32)
    pltpu.sync_copy(dst_hbm.at[wid], dst_v)
    pltpu.sync_copy(zeros_hbm.at[pl.ds(s * RPS, RPS)],
                    acc_sh.at[pl.ds(s * RPS, RPS)])
    plsc.subcore_barrier()

    DEPTH = 4

    def body(j, carry):
        pltpu.async_copy(ones_v, acc_sh.at[dst_v.at[j]], sem, add=True)

        @pl.when(j >= DEPTH)
        def _():
            pltpu.make_async_copy(zeros_hbm.at[pl.ds(0, K)], ones_v, sem).wait()

        return carry

    lax.fori_loop(0, NCHUNK, body, 0)

    def drain(j, carry):
        pltpu.make_async_copy(zeros_hbm.at[pl.ds(0, K)], ones_v, sem).wait()
        return carry

    lax.fori_loop(0, DEPTH, drain, 0)
    plsc.subcore_barrier()
    pltpu.sync_copy(acc_sh.at[pl.ds(s * RPS, RPS)],
                    out_hbm.at[c, pl.ds(s * RPS, RPS)])


assert NCHUNK % 2 == 1 and NCHUNK >= 5
TOTCH = NW * NCHUNK
CPC = TOTCH // NC
G0 = 109
G1 = 2 * NCHUNK - G0
assert G0 % 2 == 1 and G1 % 2 == 1 and G0 >= 5 and G1 >= 5
CH0 = NS * G0


@functools.partial(
    pl.kernel,
    mesh=_mesh,
    out_type=jax.ShapeDtypeStruct((NC, NP, D), jnp.float32),
    scratch_types=[
        pltpu.VMEM((2, K), jnp.int32),
        pltpu.VMEM((2, K), jnp.int32),
        pltpu.VMEM((K, D), jnp.float32),
        pltpu.VMEM((K, D), jnp.float32),
        pltpu.VMEM_SHARED((NP, D), jnp.float32),
        pltpu.SemaphoreType.DMA,
        pltpu.SemaphoreType.DMA,
        pltpu.SemaphoreType.DMA,
        pltpu.SemaphoreType.DMA,
    ],
)
def _sc_agg(x_hbm, sd_hbm, zeros_hbm, out_hbm,
            idx_a, idx_b, buf_a, buf_b, acc_sh, sem_a, sem_b, sem_ia, sem_ib):
    c = lax.axis_index("c")
    s = lax.axis_index("s")
    base = jnp.where(c == 0, s * G0, CH0 + s * G1)
    g = jnp.where(c == 0, G0, G1)
    m = (g - 3) // 2

    def wait_buf(buf, sem):
        pltpu.make_async_copy(x_hbm.at[pl.ds(0, K)], buf, sem).wait()

    def wait_idx(idx, sem):
        pltpu.make_async_copy(sd_hbm.at[0], idx, sem).wait()

    pltpu.sync_copy(sd_hbm.at[base], idx_a)
    pltpu.sync_copy(sd_hbm.at[base + 1], idx_b)
    pltpu.sync_copy(zeros_hbm.at[pl.ds(s * RPS, RPS)],
                    acc_sh.at[pl.ds(s * RPS, RPS)])
    plsc.subcore_barrier()

    pltpu.async_copy(x_hbm.at[idx_a.at[0]], buf_a, sem_a)
    pltpu.async_copy(x_hbm.at[idx_b.at[0]], buf_b, sem_b)

    def body(i, carry):
        ja = base + 2 * i
        wait_buf(buf_a, sem_a)
        pltpu.sync_copy(buf_a, acc_sh.at[idx_a.at[1]], add=True)
        pltpu.async_copy(sd_hbm.at[ja + 2], idx_a, sem_ia)
        wait_buf(buf_b, sem_b)
        pltpu.sync_copy(buf_b, acc_sh.at[idx_b.at[1]], add=True)
        wait_idx(idx_a, sem_ia)
        pltpu.async_copy(x_hbm.at[idx_a.at[0]], buf_a, sem_a)
        pltpu.async_copy(sd_hbm.at[ja + 3], idx_b, sem_ib)
        wait_idx(idx_b, sem_ib)
        pltpu.async_copy(x_hbm.at[idx_b.at[0]], buf_b, sem_b)
        return carry

    lax.fori_loop(0, m, body, 0)

    wait_buf(buf_a, sem_a)
    pltpu.sync_copy(buf_a, acc_sh.at[idx_a.at[1]], add=True)
    pltpu.async_copy(sd_hbm.at[base + g - 1], idx_a, sem_ia)
    wait_buf(buf_b, sem_b)
    pltpu.sync_copy(buf_b, acc_sh.at[idx_b.at[1]], add=True)
    wait_idx(idx_a, sem_ia)
    pltpu.async_copy(x_hbm.at[idx_a.at[0]], buf_a, sem_a)
    wait_buf(buf_a, sem_a)
    pltpu.sync_copy(buf_a, acc_sh.at[idx_a.at[1]], add=True)

    plsc.subcore_barrier()
    pltpu.sync_copy(acc_sh.at[pl.ds(s * RPS, RPS)],
                    out_hbm.at[c, pl.ds(s * RPS, RPS)])


RB = 2048


def _prep_body(deg_ref, emb_ref, xp_ref, dinv_ref):
    deg = deg_ref[0, :] + deg_ref[1, :] + 1.0
    dinv = lax.rsqrt(deg)[:, None]
    dinv_ref[...] = jnp.broadcast_to(dinv, (RB, D))
    xp_ref[...] = emb_ref[...] * dinv


_prep = pl.pallas_call(
    _prep_body,
    grid=(NP // RB,),
    in_specs=[
        pl.BlockSpec((NC, RB), lambda i: (0, i)),
        pl.BlockSpec((RB, D), lambda i: (i, 0)),
    ],
    out_specs=[
        pl.BlockSpec((RB, D), lambda i: (i, 0)),
        pl.BlockSpec((RB, D), lambda i: (i, 0)),
    ],
    out_shape=[
        jax.ShapeDtypeStruct((NP, D), jnp.float32),
        jax.ShapeDtypeStruct((NP, D), jnp.float32),
    ],
)


def _layer_body(relu, p_ref, xp_ref, dinv_ref, w_ref, b_ref, out_ref):
    z = (p_ref[0] + p_ref[1] + xp_ref[...]) * dinv_ref[...]
    h = jnp.dot(z, w_ref[...], preferred_element_type=jnp.float32) + b_ref[...]
    if relu:
        h = jnp.maximum(h, 0.0) * dinv_ref[...]
    out_ref[...] = h


def _make_layer(relu):
    return pl.pallas_call(
        functools.partial(_layer_body, relu),
        grid=(NP // RB,),
        in_specs=[
            pl.BlockSpec((NC, RB, D), lambda i: (0, i, 0)),
            pl.BlockSpec((RB, D), lambda i: (i, 0)),
            pl.BlockSpec((RB, D), lambda i: (i, 0)),
            pl.BlockSpec((D, D), lambda i: (0, 0)),
            pl.BlockSpec((1, D), lambda i: (0, 0)),
        ],
        out_specs=pl.BlockSpec((RB, D), lambda i: (i, 0)),
        out_shape=jax.ShapeDtypeStruct((NP, D), jnp.float32),
    )


_layer1 = _make_layer(True)
_layer2 = _make_layer(False)


def kernel(edge_index, emb, W1, b1, W2, b2):
    ei = edge_index.astype(jnp.int32)
    src = jnp.concatenate([ei[0], jnp.zeros((EPAD - E,), jnp.int32)])
    dst = jnp.concatenate([ei[1], jnp.full((EPAD - E,), DUMMY, jnp.int32)])
    src3 = src.reshape(NW, NCHUNK, K)
    dst3 = dst.reshape(NW, NCHUNK, K)
    sd = jnp.stack([src3, dst3], axis=2).reshape(TOTCH, 2, K)
    emb_p = jnp.pad(emb, ((0, NP - N), (0, 0)))
    z2d = jnp.zeros((NP, D), jnp.float32)
    z1d = jnp.zeros((NP,), jnp.float32)

    deg_p = _sc_deg(dst3, z1d)
    x0p, dinvb = _prep(deg_p, emb_p)
    p1 = _sc_agg(x0p, sd, z2d)
    x1p = _layer1(p1, x0p, dinvb, W1, b1.reshape(1, D))
    p2 = _sc_agg(x1p, sd, z2d)
    h2 = _layer2(p2, x1p, dinvb, W2, b2.reshape(1, D))
    return h2[:N]


# speedup vs baseline: 18.6392x; 1.1511x over previous
import functools

import jax
import jax.numpy as jnp
from jax import lax
from jax.experimental import pallas as pl
from jax.experimental.pallas import tpu as pltpu
from jax.experimental.pallas import tpu_sc as plsc

N = 10000
E = 320000
D = 128
NP = 10240
NC = 2
NS = 16
NW = NC * NS
K = 128
NCHUNK = -(-(E // NW) // K)
EPAD = NW * NCHUNK * K
DUMMY = NP - 1
RPS = NP // NS

_mesh = plsc.VectorSubcoreMesh(core_axis_name="c", subcore_axis_name="s")


@functools.partial(
    pl.kernel,
    mesh=_mesh,
    out_type=jax.ShapeDtypeStruct((NC, NP), jnp.float32),
    scratch_types=[
        pltpu.VMEM((NCHUNK, K), jnp.int32),
        pltpu.VMEM((K,), jnp.float32),
        pltpu.VMEM((RPS,), jnp.float32),
        pltpu.VMEM_SHARED((NP,), jnp.float32),
        pltpu.SemaphoreType.DMA,
    ],
)
def _sc_deg(dst_hbm, out_hbm, dst_v, ones_v, zrow_v, acc_sh, sem):
    c = lax.axis_index("c")
    s = lax.axis_index("s")
    wid = s * NC + c
    for i in range(K // 16):
        ones_v[pl.ds(i * 16, 16)] = jnp.ones((16,), jnp.float32)

    def zbody(i, carry):
        zrow_v[pl.ds(i * 16, 16)] = jnp.zeros((16,), jnp.float32)
        return carry

    lax.fori_loop(0, RPS // 16, zbody, 0)
    pltpu.sync_copy(dst_hbm.at[wid], dst_v)
    pltpu.sync_copy(zrow_v, acc_sh.at[pl.ds(s * RPS, RPS)])
    plsc.subcore_barrier()

    DEPTH = 4

    def body(j, carry):
        pltpu.async_copy(ones_v, acc_sh.at[dst_v.at[j]], sem, add=True)

        @pl.when(j >= DEPTH)
        def _():
            pltpu.make_async_copy(out_hbm.at[0, pl.ds(0, K)], ones_v, sem).wait()

        return carry

    lax.fori_loop(0, NCHUNK, body, 0)

    def drain(j, carry):
        pltpu.make_async_copy(out_hbm.at[0, pl.ds(0, K)], ones_v, sem).wait()
        return carry

    lax.fori_loop(0, DEPTH, drain, 0)
    plsc.subcore_barrier()
    pltpu.sync_copy(acc_sh.at[pl.ds(s * RPS, RPS)],
                    out_hbm.at[c, pl.ds(s * RPS, RPS)])


assert NCHUNK % 2 == 1 and NCHUNK >= 5
TOTCH = NW * NCHUNK
CPC = TOTCH // NC
G0 = 121
G1 = 2 * NCHUNK - G0
assert G0 % 2 == 1 and G1 % 2 == 1 and G0 >= 5 and G1 >= 5
CH0 = NS * G0


@functools.partial(
    pl.kernel,
    mesh=_mesh,
    out_type=jax.ShapeDtypeStruct((NC, NP, D), jnp.float32),
    scratch_types=[
        pltpu.VMEM((2, K), jnp.int32),
        pltpu.VMEM((2, K), jnp.int32),
        pltpu.VMEM((K, D), jnp.float32),
        pltpu.VMEM((K, D), jnp.float32),
        pltpu.VMEM_SHARED((NP, D), jnp.float32),
        pltpu.SemaphoreType.DMA,
        pltpu.SemaphoreType.DMA,
        pltpu.SemaphoreType.DMA,
        pltpu.SemaphoreType.DMA,
    ],
)
def _sc_agg(x_hbm, sd_hbm, out_hbm,
            idx_a, idx_b, buf_a, buf_b, acc_sh, sem_a, sem_b, sem_ia, sem_ib):
    c = lax.axis_index("c")
    s = lax.axis_index("s")
    base = jnp.where(c == 0, s * G0, CH0 + s * G1)
    g = jnp.where(c == 0, G0, G1)
    m = (g - 3) // 2

    def wait_buf(buf, sem):
        pltpu.make_async_copy(x_hbm.at[pl.ds(0, K)], buf, sem).wait()

    def wait_idx(idx, sem):
        pltpu.make_async_copy(sd_hbm.at[0], idx, sem).wait()

    def zbody(i, carry):
        for t in range(D // 16):
            buf_a[i, pl.ds(t * 16, 16)] = jnp.zeros((16,), jnp.float32)
        return carry

    lax.fori_loop(0, K, zbody, 0)
    for t in range(RPS // K):
        pltpu.sync_copy(buf_a, acc_sh.at[pl.ds(s * RPS + t * K, K)])
    pltpu.sync_copy(sd_hbm.at[base], idx_a)
    pltpu.sync_copy(sd_hbm.at[base + 1], idx_b)
    plsc.subcore_barrier()

    pltpu.async_copy(x_hbm.at[idx_a.at[0]], buf_a, sem_a)
    pltpu.async_copy(x_hbm.at[idx_b.at[0]], buf_b, sem_b)

    def body(i, carry):
        ja = base + 2 * i
        wait_buf(buf_a, sem_a)
        pltpu.sync_copy(buf_a, acc_sh.at[idx_a.at[1]], add=True)
        pltpu.async_copy(sd_hbm.at[ja + 2], idx_a, sem_ia)
        wait_buf(buf_b, sem_b)
        pltpu.sync_copy(buf_b, acc_sh.at[idx_b.at[1]], add=True)
        wait_idx(idx_a, sem_ia)
        pltpu.async_copy(x_hbm.at[idx_a.at[0]], buf_a, sem_a)
        pltpu.async_copy(sd_hbm.at[ja + 3], idx_b, sem_ib)
        wait_idx(idx_b, sem_ib)
        pltpu.async_copy(x_hbm.at[idx_b.at[0]], buf_b, sem_b)
        return carry

    lax.fori_loop(0, m, body, 0)

    wait_buf(buf_a, sem_a)
    pltpu.sync_copy(buf_a, acc_sh.at[idx_a.at[1]], add=True)
    pltpu.async_copy(sd_hbm.at[base + g - 1], idx_a, sem_ia)
    wait_buf(buf_b, sem_b)
    pltpu.sync_copy(buf_b, acc_sh.at[idx_b.at[1]], add=True)
    wait_idx(idx_a, sem_ia)
    pltpu.async_copy(x_hbm.at[idx_a.at[0]], buf_a, sem_a)
    wait_buf(buf_a, sem_a)
    pltpu.sync_copy(buf_a, acc_sh.at[idx_a.at[1]], add=True)

    plsc.subcore_barrier()
    pltpu.sync_copy(acc_sh.at[pl.ds(s * RPS, RPS)],
                    out_hbm.at[c, pl.ds(s * RPS, RPS)])


RB = 2048


def _prep_body(deg_ref, emb_ref, xp_ref, dinv_ref):
    deg = deg_ref[0, :] + deg_ref[1, :] + 1.0
    dinv = lax.rsqrt(deg)[:, None]
    dinv_ref[...] = jnp.broadcast_to(dinv, (RB, D))
    xp_ref[...] = emb_ref[...] * dinv


_prep = pl.pallas_call(
    _prep_body,
    grid=(NP // RB,),
    in_specs=[
        pl.BlockSpec((NC, RB), lambda i: (0, i)),
        pl.BlockSpec((RB, D), lambda i: (i, 0)),
    ],
    out_specs=[
        pl.BlockSpec((RB, D), lambda i: (i, 0)),
        pl.BlockSpec((RB, D), lambda i: (i, 0)),
    ],
    out_shape=[
        jax.ShapeDtypeStruct((NP, D), jnp.float32),
        jax.ShapeDtypeStruct((NP, D), jnp.float32),
    ],
)


def _layer_body(relu, p_ref, xp_ref, dinv_ref, w_ref, b_ref, out_ref):
    z = (p_ref[0] + p_ref[1] + xp_ref[...]) * dinv_ref[...]
    h = jnp.dot(z, w_ref[...], preferred_element_type=jnp.float32) + b_ref[...]
    if relu:
        h = jnp.maximum(h, 0.0) * dinv_ref[...]
    out_ref[...] = h


def _make_layer(relu):
    return pl.pallas_call(
        functools.partial(_layer_body, relu),
        grid=(NP // RB,),
        in_specs=[
            pl.BlockSpec((NC, RB, D), lambda i: (0, i, 0)),
            pl.BlockSpec((RB, D), lambda i: (i, 0)),
            pl.BlockSpec((RB, D), lambda i: (i, 0)),
            pl.BlockSpec((D, D), lambda i: (0, 0)),
            pl.BlockSpec((1, D), lambda i: (0, 0)),
        ],
        out_specs=pl.BlockSpec((RB, D), lambda i: (i, 0)),
        out_shape=jax.ShapeDtypeStruct((NP, D), jnp.float32),
    )


_layer1 = _make_layer(True)
_layer2 = _make_layer(False)


def kernel(edge_index, emb, W1, b1, W2, b2):
    ei = edge_index.astype(jnp.int32)
    src = jnp.concatenate([ei[0], jnp.zeros((EPAD - E,), jnp.int32)])
    dst = jnp.concatenate([ei[1], jnp.full((EPAD - E,), DUMMY, jnp.int32)])
    src3 = src.reshape(NW, NCHUNK, K)
    dst3 = dst.reshape(NW, NCHUNK, K)
    sd = jnp.stack([src3, dst3], axis=2).reshape(TOTCH, 2, K)
    emb_p = jnp.pad(emb, ((0, NP - N), (0, 0)))

    deg_p = _sc_deg(dst3)
    x0p, dinvb = _prep(deg_p, emb_p)
    p1 = _sc_agg(x0p, sd)
    x1p = _layer1(p1, x0p, dinvb, W1, b1.reshape(1, D))
    p2 = _sc_agg(x1p, sd)
    h2 = _layer2(p2, x1p, dinvb, W2, b2.reshape(1, D))
    return h2[:N]


# speedup vs baseline: 30.4653x; 1.6345x over previous
import functools

import jax
import jax.numpy as jnp
from jax import lax
from jax.experimental import pallas as pl
from jax.experimental.pallas import tpu as pltpu
from jax.experimental.pallas import tpu_sc as plsc

N = 10000
E = 320000
D = 128
NP = 10240
NC = 2
NS = 16
NW = NC * NS
K = 128
NCHUNK = 80
EPAD = NW * NCHUNK * K
DUMMY = NP - 1
RPS = NP // NS

_mesh = plsc.VectorSubcoreMesh(core_axis_name="c", subcore_axis_name="s")


@functools.partial(
    pl.kernel,
    mesh=_mesh,
    out_type=jax.ShapeDtypeStruct((NC, NP), jnp.float32),
    scratch_types=[
        pltpu.VMEM((NCHUNK, K), jnp.int32),
        pltpu.VMEM((K,), jnp.float32),
        pltpu.VMEM((RPS,), jnp.float32),
        pltpu.VMEM_SHARED((NP,), jnp.float32),
        pltpu.SemaphoreType.DMA,
    ],
)
def _sc_deg(dst_hbm, out_hbm, dst_v, ones_v, zrow_v, acc_sh, sem):
    c = lax.axis_index("c")
    s = lax.axis_index("s")
    wid = s * NC + c
    for i in range(K // 16):
        ones_v[pl.ds(i * 16, 16)] = jnp.ones((16,), jnp.float32)

    pltpu.async_copy(dst_hbm.at[wid], dst_v, sem)

    def zbody(i, carry):
        zrow_v[pl.ds(i * 16, 16)] = jnp.zeros((16,), jnp.float32)
        return carry

    lax.fori_loop(0, RPS // 16, zbody, 0)
    pltpu.sync_copy(zrow_v, acc_sh.at[pl.ds(s * RPS, RPS)])
    pltpu.make_async_copy(dst_hbm.at[0], dst_v, sem).wait()
    plsc.subcore_barrier()

    DEPTH = 6

    def body(j, carry):
        pltpu.async_copy(ones_v, acc_sh.at[dst_v.at[j]], sem, add=True)

        @pl.when(j >= DEPTH)
        def _():
            pltpu.make_async_copy(out_hbm.at[0, pl.ds(0, K)], ones_v, sem).wait()

        return carry

    lax.fori_loop(0, NCHUNK, body, 0)

    def drain(j, carry):
        pltpu.make_async_copy(out_hbm.at[0, pl.ds(0, K)], ones_v, sem).wait()
        return carry

    lax.fori_loop(0, DEPTH, drain, 0)
    plsc.subcore_barrier()
    pltpu.sync_copy(acc_sh.at[pl.ds(s * RPS, RPS)],
                    out_hbm.at[c, pl.ds(s * RPS, RPS)])


KA = 88
TOTCH = 3648
EPADA = TOTCH * KA
G0 = 148
G1 = TOTCH // NS - G0
assert G0 % 4 == 0 and G1 % 4 == 0 and G0 >= 8 and G1 >= 8
CH0 = NS * G0


@functools.partial(
    pl.kernel,
    mesh=_mesh,
    out_type=jax.ShapeDtypeStruct((NC, NP, D), jnp.float32),
    scratch_types=(
        [pltpu.VMEM((2, KA), jnp.int32)] * 4
        + [pltpu.VMEM((KA, D), jnp.float32)] * 4
        + [pltpu.VMEM_SHARED((NP, D), jnp.float32)]
        + [pltpu.SemaphoreType.DMA] * 8
    ),
)
def _sc_agg(x_hbm, sd_hbm, out_hbm,
            i0, i1, i2, i3, b0, b1, b2, b3, acc_sh,
            g0s, g1s, g2s, g3s, s0s, s1s, s2s, s3s):
    c = lax.axis_index("c")
    s = lax.axis_index("s")
    base = jnp.where(c == 0, s * G0, CH0 + s * G1)
    g = jnp.where(c == 0, G0, G1)
    m = g // 4 - 1
    idx = [i0, i1, i2, i3]
    buf = [b0, b1, b2, b3]
    gsem = [g0s, g1s, g2s, g3s]
    isem = [s0s, s1s, s2s, s3s]

    def wait_buf(t):
        pltpu.make_async_copy(x_hbm.at[pl.ds(0, KA)], buf[t], gsem[t]).wait()

    def wait_idx(t):
        pltpu.make_async_copy(sd_hbm.at[0], idx[t], isem[t]).wait()

    def zbody(i, carry):
        for t in range(D // 16):
            b3[i, pl.ds(t * 16, 16)] = jnp.zeros((16,), jnp.float32)
        return carry

    for t in range(4):
        pltpu.async_copy(sd_hbm.at[base + t], idx[t], isem[t])
    lax.fori_loop(0, KA, zbody, 0)
    for t in range(3):
        wait_idx(t)
        pltpu.async_copy(x_hbm.at[idx[t].at[0]], buf[t], gsem[t])
    for t in range(RPS // KA):
        pltpu.sync_copy(b3, acc_sh.at[pl.ds(s * RPS + t * KA, KA)])
    _REM = RPS - (RPS // KA) * KA
    if _REM:
        pltpu.sync_copy(b3.at[pl.ds(0, _REM)],
                        acc_sh.at[pl.ds(s * RPS + (RPS // KA) * KA, _REM)])
    wait_idx(3)
    pltpu.async_copy(x_hbm.at[idx[3].at[0]], b3, g3s)
    plsc.subcore_barrier()

    def body(i, carry):
        j = base + 4 * i
        wait_buf(0)
        pltpu.sync_copy(b0, acc_sh.at[i0.at[1]], add=True)
        pltpu.async_copy(sd_hbm.at[j + 4], i0, s0s)
        wait_buf(1)
        pltpu.sync_copy(b1, acc_sh.at[i1.at[1]], add=True)
        pltpu.async_copy(sd_hbm.at[j + 5], i1, s1s)
        wait_idx(0)
        pltpu.async_copy(x_hbm.at[i0.at[0]], b0, g0s)
        wait_buf(2)
        pltpu.sync_copy(b2, acc_sh.at[i2.at[1]], add=True)
        pltpu.async_copy(sd_hbm.at[j + 6], i2, s2s)
        wait_idx(1)
        pltpu.async_copy(x_hbm.at[i1.at[0]], b1, g1s)
        wait_buf(3)
        pltpu.sync_copy(b3, acc_sh.at[i3.at[1]], add=True)
        pltpu.async_copy(sd_hbm.at[j + 7], i3, s3s)
        wait_idx(2)
        pltpu.async_copy(x_hbm.at[i2.at[0]], b2, g2s)
        wait_idx(3)
        pltpu.async_copy(x_hbm.at[i3.at[0]], b3, g3s)
        return carry

    lax.fori_loop(0, m, body, 0)

    for t in range(4):
        wait_buf(t)
        pltpu.sync_copy(buf[t], acc_sh.at[idx[t].at[1]], add=True)

    plsc.subcore_barrier()
    pltpu.sync_copy(acc_sh.at[pl.ds(s * RPS, RPS)],
                    out_hbm.at[c, pl.ds(s * RPS, RPS)])


RB = 2048


def _prep_body(deg_ref, emb_ref, xp_ref, dinv_ref):
    deg = deg_ref[0, :] + deg_ref[1, :] + 1.0
    dinv = lax.rsqrt(deg)[:, None]
    dinv_ref[...] = jnp.broadcast_to(dinv, (RB, D))
    xp_ref[...] = emb_ref[...] * dinv


_prep = pl.pallas_call(
    _prep_body,
    grid=(NP // RB,),
    in_specs=[
        pl.BlockSpec((NC, RB), lambda i: (0, i)),
        pl.BlockSpec((RB, D), lambda i: (i, 0)),
    ],
    out_specs=[
        pl.BlockSpec((RB, D), lambda i: (i, 0)),
        pl.BlockSpec((RB, D), lambda i: (i, 0)),
    ],
    out_shape=[
        jax.ShapeDtypeStruct((NP, D), jnp.float32),
        jax.ShapeDtypeStruct((NP, D), jnp.float32),
    ],
)


def _layer_body(relu, p_ref, xp_ref, dinv_ref, w_ref, b_ref, out_ref):
    z = (p_ref[0] + p_ref[1] + xp_ref[...]) * dinv_ref[...]
    h = jnp.dot(z, w_ref[...], preferred_element_type=jnp.float32) + b_ref[...]
    if relu:
        h = jnp.maximum(h, 0.0) * dinv_ref[...]
    out_ref[...] = h


def _make_layer(relu):
    return pl.pallas_call(
        functools.partial(_layer_body, relu),
        grid=(NP // RB,),
        in_specs=[
            pl.BlockSpec((NC, RB, D), lambda i: (0, i, 0)),
            pl.BlockSpec((RB, D), lambda i: (i, 0)),
            pl.BlockSpec((RB, D), lambda i: (i, 0)),
            pl.BlockSpec((D, D), lambda i: (0, 0)),
            pl.BlockSpec((1, D), lambda i: (0, 0)),
        ],
        out_specs=pl.BlockSpec((RB, D), lambda i: (i, 0)),
        out_shape=jax.ShapeDtypeStruct((NP, D), jnp.float32),
    )


_layer1 = _make_layer(True)
_layer2 = _make_layer(False)


def kernel(edge_index, emb, W1, b1, W2, b2):
    ei = edge_index.astype(jnp.int32)
    dstd = jnp.concatenate([ei[1], jnp.full((EPAD - E,), DUMMY, jnp.int32)])
    dst3 = dstd.reshape(NW, NCHUNK, K)
    srca = jnp.concatenate([ei[0], jnp.zeros((EPADA - E,), jnp.int32)])
    dsta = jnp.concatenate([ei[1], jnp.full((EPADA - E,), DUMMY, jnp.int32)])
    sd = jnp.stack([srca.reshape(TOTCH, KA), dsta.reshape(TOTCH, KA)],
                   axis=1)
    emb_p = jnp.pad(emb, ((0, NP - N), (0, 0)))

    deg_p = _sc_deg(dst3)
    x0p, dinvb = _prep(deg_p, emb_p)
    p1 = _sc_agg(x0p, sd)
    x1p = _layer1(p1, x0p, dinvb, W1, b1.reshape(1, D))
    p2 = _sc_agg(x1p, sd)
    h2 = _layer2(p2, x1p, dinvb, W2, b2.reshape(1, D))
    return h2[:N]
